# baseline (speedup 1.0000x reference)
import functools

import jax
import jax.numpy as jnp
from jax.experimental import pallas as pl
from jax.experimental.pallas import tpu as pltpu

_B = 2
_N_SURF = 3000
_N_NM = 1000
_N = _N_SURF + _N_NM
_NP = 4096
_HEADS = 4
_HID = 64
_FEAT = _HEADS * _HID
_OUT_CH = 2
_PBLK = 512
_DBLK = 512
_SBLK = 128
_NDB = _NP // _DBLK
_NSB = _NP // _SBLK
_RADIUS = 0.05
_RADIUS_SQ = float(0.0025)
_TH = 0.17
_PAD_VAL = 100.0


def _head_logits(h, aw_s, aw_d):
    acols, dcols = [], []
    for hd in range(_HEADS):
        hs = h[:, hd * _HID:(hd + 1) * _HID]
        acols.append(jnp.sum(hs * aw_s[hd:hd + 1, :], axis=1, keepdims=True))
        dcols.append(jnp.sum(hs * aw_d[hd:hd + 1, :], axis=1, keepdims=True))
    return (jnp.concatenate(acols, axis=1),
            jnp.concatenate(dcols, axis=1))


def _proj_body(x_ref, w_ref, aws_ref, awd_ref, h_ref, as_ref, ad_ref):
    x = x_ref[0]
    w = w_ref[...]
    h = jnp.dot(x, w, preferred_element_type=jnp.float32)
    h_ref[0] = h
    as_ref[0], ad_ref[0] = _head_logits(h, aws_ref[...], awd_ref[...])


def _attn_body(lo_ref, hi_ref, posdt_ref, posnd_ref, h_ref, as_ref, adt_ref,
               b_ref, w_ref, p0_ref, p1_ref, *o_refs, fuse_fc):
    b = pl.program_id(0)
    j = pl.program_id(1)
    pos_dt = posdt_ref[0]
    sq_d = jnp.sum(pos_dt * pos_dt, axis=0, keepdims=True)
    ad_row = adt_ref[0]
    lo = lo_ref[b, j]
    hi = hi_ref[b, j]

    def chunk_mask(k):
        off = k * _SBLK
        pos_s = posnd_ref[0, pl.ds(off, _SBLK), :]
        sq_s = jnp.sum(pos_s * pos_s, axis=1, keepdims=True)
        dots = jnp.dot(pos_s, pos_dt, preferred_element_type=jnp.float32)
        d2 = sq_s + sq_d - 2.0 * dots
        return d2 < _RADIUS_SQ

    def masked_exp(k, mask):
        as_chunk = as_ref[0, pl.ds(k * _SBLK, _SBLK), :]
        exs = []
        for hd in range(_HEADS):
            e = as_chunk[:, hd:hd + 1] + ad_row[hd:hd + 1, :]
            e = jnp.where(e >= 0, e, 0.2 * e)
            exs.append(jnp.where(mask, jnp.exp(e), 0.0))
        return exs

    def den_chunk(k, carry):
        ls, lo1, hi1 = carry
        mask = chunk_mask(k)
        live = jnp.max(mask.astype(jnp.float32)) > 0.0
        nls = jax.lax.cond(
            live,
            lambda: tuple(ls[hd] + jnp.sum(masked_exp(k, mask)[hd],
                                           axis=0, keepdims=True)
                          for hd in range(_HEADS)),
            lambda: ls)
        lo1 = jnp.where(live, jnp.minimum(lo1, k), lo1)
        hi1 = jnp.where(live, jnp.maximum(hi1, k), hi1)
        return nls, lo1, hi1

    ls, lo1, hi1 = jax.lax.fori_loop(
        lo, hi + 1, den_chunk,
        (tuple(jnp.zeros((1, _DBLK), jnp.float32) for _ in range(_HEADS)),
         jnp.int32(_NSB), jnp.int32(-1)))
    dens = [ls[hd] + 1e-16 for hd in range(_HEADS)]

    def agg_chunk(k, accs):
        exs = masked_exp(k, chunk_mask(k))
        off = k * _SBLK
        h_chunk = h_ref[0, pl.ds(off, _SBLK), :]
        na = []
        for hd in range(_HEADS):
            alpha = exs[hd] / dens[hd]
            hc = h_chunk[:, hd * _HID:(hd + 1) * _HID]
            contrib = jax.lax.dot_general(
                hc, alpha, (((0,), (0,)), ((), ())),
                preferred_element_type=jnp.float32)
            na.append(accs[hd] + contrib)
        return tuple(na)

    accs = jax.lax.fori_loop(
        lo1, hi1 + 1, agg_chunk,
        tuple(jnp.zeros((_HID, _DBLK), jnp.float32) for _ in range(_HEADS)))

    out = jnp.concatenate(
        [jnp.transpose(accs[hd]) for hd in range(_HEADS)],
        axis=1)
    out = jnp.maximum(out + b_ref[...], 0.0)
    if fuse_fc:
        y = jnp.dot(out, w_ref[...],
                    preferred_element_type=jnp.float32) + p0_ref[...]
        o_refs[0][0] = y
    else:
        h2 = jnp.dot(out, w_ref[...], preferred_element_type=jnp.float32)
        o_refs[0][0] = h2
        o_refs[1][0], o_refs[2][0] = _head_logits(h2, p0_ref[...], p1_ref[...])


def _proj(x, w, aw_s, aw_d):
    cin = x.shape[-1]
    return pl.pallas_call(
        _proj_body,
        grid=(_B, _NP // _PBLK),
        in_specs=[
            pl.BlockSpec((1, _PBLK, cin), lambda b, j: (b, j, 0)),
            pl.BlockSpec((cin, _FEAT), lambda b, j: (0, 0)),
            pl.BlockSpec((_HEADS, _HID), lambda b, j: (0, 0)),
            pl.BlockSpec((_HEADS, _HID), lambda b, j: (0, 0)),
        ],
        out_specs=[
            pl.BlockSpec((1, _PBLK, _FEAT), lambda b, j: (b, j, 0)),
            pl.BlockSpec((1, _PBLK, _HEADS), lambda b, j: (b, j, 0)),
            pl.BlockSpec((1, _PBLK, _HEADS), lambda b, j: (b, j, 0)),
        ],
        out_shape=[
            jax.ShapeDtypeStruct((_B, _NP, _FEAT), jnp.float32),
            jax.ShapeDtypeStruct((_B, _NP, _HEADS), jnp.float32),
            jax.ShapeDtypeStruct((_B, _NP, _HEADS), jnp.float32),
        ],
        compiler_params=pltpu.CompilerParams(
            dimension_semantics=("parallel", "parallel")),
    )(x, w, aw_s, aw_d)


def _attn(lo, hi, pos_t, pos_nd, h, as_, ad_t, bias, w, p0, p1, fuse_fc):
    if fuse_fc:
        wshape = (_FEAT, _OUT_CH)
        p0shape = (1, _OUT_CH)
        p1shape = (1, _OUT_CH)
        out_specs = pl.BlockSpec((1, _DBLK, _OUT_CH),
                                 lambda b, j, lo, hi: (b, j, 0))
        out_shape = jax.ShapeDtypeStruct((_B, _NP, _OUT_CH), jnp.float32)
    else:
        wshape = (_FEAT, _FEAT)
        p0shape = (_HEADS, _HID)
        p1shape = (_HEADS, _HID)
        out_specs = [
            pl.BlockSpec((1, _DBLK, _FEAT), lambda b, j, lo, hi: (b, j, 0)),
            pl.BlockSpec((1, _DBLK, _HEADS), lambda b, j, lo, hi: (b, j, 0)),
            pl.BlockSpec((1, _DBLK, _HEADS), lambda b, j, lo, hi: (b, j, 0)),
        ]
        out_shape = [
            jax.ShapeDtypeStruct((_B, _NP, _FEAT), jnp.float32),
            jax.ShapeDtypeStruct((_B, _NP, _HEADS), jnp.float32),
            jax.ShapeDtypeStruct((_B, _NP, _HEADS), jnp.float32),
        ]
    return pl.pallas_call(
        functools.partial(_attn_body, fuse_fc=fuse_fc),
        grid_spec=pltpu.PrefetchScalarGridSpec(
            num_scalar_prefetch=2,
            grid=(_B, _NDB),
            in_specs=[
                pl.BlockSpec((1, 3, _DBLK), lambda b, j, lo, hi: (b, 0, j)),
                pl.BlockSpec((1, _NP, 3), lambda b, j, lo, hi: (b, 0, 0)),
                pl.BlockSpec((1, _NP, _FEAT), lambda b, j, lo, hi: (b, 0, 0)),
                pl.BlockSpec((1, _NP, _HEADS), lambda b, j, lo, hi: (b, 0, 0)),
                pl.BlockSpec((1, _HEADS, _DBLK), lambda b, j, lo, hi: (b, 0, j)),
                pl.BlockSpec((1, _FEAT), lambda b, j, lo, hi: (0, 0)),
                pl.BlockSpec(wshape, lambda b, j, lo, hi: (0, 0)),
                pl.BlockSpec(p0shape, lambda b, j, lo, hi: (0, 0)),
                pl.BlockSpec(p1shape, lambda b, j, lo, hi: (0, 0)),
            ],
            out_specs=out_specs,
        ),
        out_shape=out_shape,
        compiler_params=pltpu.CompilerParams(
            dimension_semantics=("parallel", "arbitrary")),
    )(lo, hi, pos_t, pos_nd, h, as_, ad_t, bias, w, p0, p1)


def _chunk_ranges(pos_t):
    xp = pos_t[:, 0, :]
    cs_min = xp[:, 0::_SBLK]
    cs_max = xp[:, _SBLK - 1::_SBLK]
    xd_min = xp[:, 0::_DBLK]
    xd_max = xp[:, _DBLK - 1::_DBLK]
    lo = jax.vmap(lambda a, v: jnp.searchsorted(a, v, side='left'))(
        cs_max, xd_min - _TH).astype(jnp.int32)
    hi = (jax.vmap(lambda a, v: jnp.searchsorted(a, v, side='right'))(
        cs_min, xd_max + _TH) - 1).astype(jnp.int32)
    return lo, hi


def kernel(pos, pos_non_manifold, W1, a1_src, a1_dst, b1,
           W2, a2_src, a2_dst, b2, W_fc, b_fc):
    pos_t = jnp.concatenate([pos, pos_non_manifold], axis=2)
    perm = jnp.argsort(pos_t[:, 0, :], axis=1)
    inv = jnp.argsort(perm, axis=1)
    pos_t = jnp.take_along_axis(pos_t, perm[:, None, :], axis=2)
    pos_t = jnp.pad(pos_t, ((0, 0), (0, 0), (0, _NP - _N)),
                    constant_values=_PAD_VAL)
    pos_nd = pos_t.transpose(0, 2, 1)
    pos_nd0 = pos_nd.at[:, _N:, :].set(0.0)
    lo, hi = _chunk_ranges(pos_t)

    b1r = b1.reshape(1, _FEAT)
    b2r = b2.reshape(1, _FEAT)
    bfc = b_fc.reshape(1, _OUT_CH)

    h, as_, ad = _proj(pos_nd0, W1, a1_src, a1_dst)
    h2, as2, ad2 = _attn(lo, hi, pos_t, pos_nd, h, as_,
                         ad.transpose(0, 2, 1), b1r, W2,
                         a2_src, a2_dst, fuse_fc=False)
    y = _attn(lo, hi, pos_t, pos_nd, h2, as2, ad2.transpose(0, 2, 1),
              b2r, W_fc, bfc, bfc, fuse_fc=True)

    y = jnp.take_along_axis(y[:, :_N], inv[:, :, None], axis=1)
    return y[:, _N_NM:_N].reshape(_B, _OUT_CH, _N_SURF)


# speedup vs baseline: 1.1017x; 1.1017x over previous
import functools

import jax
import jax.numpy as jnp
from jax.experimental import pallas as pl
from jax.experimental.pallas import tpu as pltpu

_B = 2
_N_SURF = 3000
_N_NM = 1000
_N = _N_SURF + _N_NM
_NP = 4096
_HEADS = 4
_HID = 64
_FEAT = _HEADS * _HID
_OUT_CH = 2
_PBLK = 512
_DBLK = 512
_SBLK = 256
_NDB = _NP // _DBLK
_NSB = _NP // _SBLK
_RADIUS = 0.05
_RADIUS_SQ = float(0.0025)
_TH = 0.17
_PAD_VAL = 100.0


def _head_logits(h, aw_s, aw_d):
    acols, dcols = [], []
    for hd in range(_HEADS):
        hs = h[:, hd * _HID:(hd + 1) * _HID]
        acols.append(jnp.sum(hs * aw_s[hd:hd + 1, :], axis=1, keepdims=True))
        dcols.append(jnp.sum(hs * aw_d[hd:hd + 1, :], axis=1, keepdims=True))
    return (jnp.concatenate(acols, axis=1),
            jnp.concatenate(dcols, axis=1))


def _proj_body(x_ref, w_ref, aws_ref, awd_ref, h_ref, as_ref, ad_ref):
    x = x_ref[0]
    w = w_ref[...]
    h = jnp.dot(x, w, preferred_element_type=jnp.float32)
    h_ref[0] = h
    as_ref[0], ad_ref[0] = _head_logits(h, aws_ref[...], awd_ref[...])


def _attn_body(lo_ref, hi_ref, posdt_ref, posnd_ref, h_ref, as_ref, adt_ref,
               b_ref, w_ref, p0_ref, p1_ref, *o_refs, fuse_fc):
    b = pl.program_id(0)
    j = pl.program_id(1)
    pos_dt = posdt_ref[0]
    sq_d = jnp.sum(pos_dt * pos_dt, axis=0, keepdims=True)
    ad_row = adt_ref[0]
    lo = lo_ref[b, j]
    hi = hi_ref[b, j]

    def chunk_mask(k):
        off = k * _SBLK
        pos_s = posnd_ref[0, pl.ds(off, _SBLK), :]
        sq_s = jnp.sum(pos_s * pos_s, axis=1, keepdims=True)
        dots = jnp.dot(pos_s, pos_dt, preferred_element_type=jnp.float32)
        d2 = sq_s + sq_d - 2.0 * dots
        return d2 < _RADIUS_SQ

    def masked_exp(k, mask):
        as_chunk = as_ref[0, pl.ds(k * _SBLK, _SBLK), :]
        exs = []
        for hd in range(_HEADS):
            e = as_chunk[:, hd:hd + 1] + ad_row[hd:hd + 1, :]
            e = jnp.where(e >= 0, e, 0.2 * e)
            exs.append(jnp.where(mask, jnp.exp(e), 0.0))
        return exs

    def den_chunk(k, carry):
        ls, lo1, hi1 = carry
        mask = chunk_mask(k)
        live = jnp.max(mask.astype(jnp.float32)) > 0.0
        exs = masked_exp(k, mask)
        nls = tuple(ls[hd] + jnp.sum(exs[hd], axis=0, keepdims=True)
                    for hd in range(_HEADS))
        lo1 = jnp.where(live, jnp.minimum(lo1, k), lo1)
        hi1 = jnp.where(live, jnp.maximum(hi1, k), hi1)
        return nls, lo1, hi1

    ls, lo1, hi1 = jax.lax.fori_loop(
        lo, hi + 1, den_chunk,
        (tuple(jnp.zeros((1, _DBLK), jnp.float32) for _ in range(_HEADS)),
         jnp.int32(_NSB), jnp.int32(-1)))
    dens = [ls[hd] + 1e-16 for hd in range(_HEADS)]

    def agg_chunk(k, accs):
        exs = masked_exp(k, chunk_mask(k))
        off = k * _SBLK
        h_chunk = h_ref[0, pl.ds(off, _SBLK), :]
        na = []
        for hd in range(_HEADS):
            alpha = exs[hd] / dens[hd]
            hc = h_chunk[:, hd * _HID:(hd + 1) * _HID]
            contrib = jax.lax.dot_general(
                hc, alpha, (((0,), (0,)), ((), ())),
                preferred_element_type=jnp.float32)
            na.append(accs[hd] + contrib)
        return tuple(na)

    accs = jax.lax.fori_loop(
        lo1, hi1 + 1, agg_chunk,
        tuple(jnp.zeros((_HID, _DBLK), jnp.float32) for _ in range(_HEADS)))

    out = jnp.concatenate(
        [jnp.transpose(accs[hd]) for hd in range(_HEADS)],
        axis=1)
    out = jnp.maximum(out + b_ref[...], 0.0)
    if fuse_fc:
        y = jnp.dot(out, w_ref[...],
                    preferred_element_type=jnp.float32) + p0_ref[...]
        o_refs[0][0] = y
    else:
        h2 = jnp.dot(out, w_ref[...], preferred_element_type=jnp.float32)
        o_refs[0][0] = h2
        o_refs[1][0], o_refs[2][0] = _head_logits(h2, p0_ref[...], p1_ref[...])


def _proj(x, w, aw_s, aw_d):
    cin = x.shape[-1]
    return pl.pallas_call(
        _proj_body,
        grid=(_B, _NP // _PBLK),
        in_specs=[
            pl.BlockSpec((1, _PBLK, cin), lambda b, j: (b, j, 0)),
            pl.BlockSpec((cin, _FEAT), lambda b, j: (0, 0)),
            pl.BlockSpec((_HEADS, _HID), lambda b, j: (0, 0)),
            pl.BlockSpec((_HEADS, _HID), lambda b, j: (0, 0)),
        ],
        out_specs=[
            pl.BlockSpec((1, _PBLK, _FEAT), lambda b, j: (b, j, 0)),
            pl.BlockSpec((1, _PBLK, _HEADS), lambda b, j: (b, j, 0)),
            pl.BlockSpec((1, _PBLK, _HEADS), lambda b, j: (b, j, 0)),
        ],
        out_shape=[
            jax.ShapeDtypeStruct((_B, _NP, _FEAT), jnp.float32),
            jax.ShapeDtypeStruct((_B, _NP, _HEADS), jnp.float32),
            jax.ShapeDtypeStruct((_B, _NP, _HEADS), jnp.float32),
        ],
        compiler_params=pltpu.CompilerParams(
            dimension_semantics=("parallel", "parallel")),
    )(x, w, aw_s, aw_d)


def _attn(lo, hi, pos_t, pos_nd, h, as_, ad_t, bias, w, p0, p1, fuse_fc):
    if fuse_fc:
        wshape = (_FEAT, _OUT_CH)
        p0shape = (1, _OUT_CH)
        p1shape = (1, _OUT_CH)
        out_specs = pl.BlockSpec((1, _DBLK, _OUT_CH),
                                 lambda b, j, lo, hi: (b, j, 0))
        out_shape = jax.ShapeDtypeStruct((_B, _NP, _OUT_CH), jnp.float32)
    else:
        wshape = (_FEAT, _FEAT)
        p0shape = (_HEADS, _HID)
        p1shape = (_HEADS, _HID)
        out_specs = [
            pl.BlockSpec((1, _DBLK, _FEAT), lambda b, j, lo, hi: (b, j, 0)),
            pl.BlockSpec((1, _DBLK, _HEADS), lambda b, j, lo, hi: (b, j, 0)),
            pl.BlockSpec((1, _DBLK, _HEADS), lambda b, j, lo, hi: (b, j, 0)),
        ]
        out_shape = [
            jax.ShapeDtypeStruct((_B, _NP, _FEAT), jnp.float32),
            jax.ShapeDtypeStruct((_B, _NP, _HEADS), jnp.float32),
            jax.ShapeDtypeStruct((_B, _NP, _HEADS), jnp.float32),
        ]
    return pl.pallas_call(
        functools.partial(_attn_body, fuse_fc=fuse_fc),
        grid_spec=pltpu.PrefetchScalarGridSpec(
            num_scalar_prefetch=2,
            grid=(_B, _NDB),
            in_specs=[
                pl.BlockSpec((1, 3, _DBLK), lambda b, j, lo, hi: (b, 0, j)),
                pl.BlockSpec((1, _NP, 3), lambda b, j, lo, hi: (b, 0, 0)),
                pl.BlockSpec((1, _NP, _FEAT), lambda b, j, lo, hi: (b, 0, 0)),
                pl.BlockSpec((1, _NP, _HEADS), lambda b, j, lo, hi: (b, 0, 0)),
                pl.BlockSpec((1, _HEADS, _DBLK), lambda b, j, lo, hi: (b, 0, j)),
                pl.BlockSpec((1, _FEAT), lambda b, j, lo, hi: (0, 0)),
                pl.BlockSpec(wshape, lambda b, j, lo, hi: (0, 0)),
                pl.BlockSpec(p0shape, lambda b, j, lo, hi: (0, 0)),
                pl.BlockSpec(p1shape, lambda b, j, lo, hi: (0, 0)),
            ],
            out_specs=out_specs,
        ),
        out_shape=out_shape,
        compiler_params=pltpu.CompilerParams(
            dimension_semantics=("parallel", "arbitrary")),
    )(lo, hi, pos_t, pos_nd, h, as_, ad_t, bias, w, p0, p1)


def _chunk_ranges(pos_t):
    xp = pos_t[:, 0, :]
    cs_min = xp[:, 0::_SBLK]
    cs_max = xp[:, _SBLK - 1::_SBLK]
    xd_min = xp[:, 0::_DBLK]
    xd_max = xp[:, _DBLK - 1::_DBLK]
    lo = jax.vmap(lambda a, v: jnp.searchsorted(a, v, side='left'))(
        cs_max, xd_min - _TH).astype(jnp.int32)
    hi = (jax.vmap(lambda a, v: jnp.searchsorted(a, v, side='right'))(
        cs_min, xd_max + _TH) - 1).astype(jnp.int32)
    return lo, hi


def kernel(pos, pos_non_manifold, W1, a1_src, a1_dst, b1,
           W2, a2_src, a2_dst, b2, W_fc, b_fc):
    pos_t = jnp.concatenate([pos, pos_non_manifold], axis=2)
    perm = jnp.argsort(pos_t[:, 0, :], axis=1)
    inv = jnp.argsort(perm, axis=1)
    pos_t = jnp.take_along_axis(pos_t, perm[:, None, :], axis=2)
    pos_t = jnp.pad(pos_t, ((0, 0), (0, 0), (0, _NP - _N)),
                    constant_values=_PAD_VAL)
    pos_nd = pos_t.transpose(0, 2, 1)
    pos_nd0 = pos_nd.at[:, _N:, :].set(0.0)
    lo, hi = _chunk_ranges(pos_t)

    b1r = b1.reshape(1, _FEAT)
    b2r = b2.reshape(1, _FEAT)
    bfc = b_fc.reshape(1, _OUT_CH)

    h, as_, ad = _proj(pos_nd0, W1, a1_src, a1_dst)
    h2, as2, ad2 = _attn(lo, hi, pos_t, pos_nd, h, as_,
                         ad.transpose(0, 2, 1), b1r, W2,
                         a2_src, a2_dst, fuse_fc=False)
    y = _attn(lo, hi, pos_t, pos_nd, h2, as2, ad2.transpose(0, 2, 1),
              b2r, W_fc, bfc, bfc, fuse_fc=True)

    y = jnp.take_along_axis(y[:, :_N], inv[:, :, None], axis=1)
    return y[:, _N_NM:_N].reshape(_B, _OUT_CH, _N_SURF)


# speedup vs baseline: 1.1720x; 1.0638x over previous
import functools

import jax
import jax.numpy as jnp
from jax.experimental import pallas as pl
from jax.experimental.pallas import tpu as pltpu

_B = 2
_N_SURF = 3000
_N_NM = 1000
_N = _N_SURF + _N_NM
_NP = 4096
_HEADS = 4
_HID = 64
_FEAT = _HEADS * _HID
_OUT_CH = 2
_PBLK = 512
_DBLK = 512
_SBLK = 256
_NDB = _NP // _DBLK
_NSB = _NP // _SBLK
_RADIUS = 0.05
_RADIUS_SQ = float(0.0025)
_TH = 0.17
_PAD_VAL = 100.0


def _head_logits(h, aw_s, aw_d):
    acols, dcols = [], []
    for hd in range(_HEADS):
        hs = h[:, hd * _HID:(hd + 1) * _HID]
        acols.append(jnp.sum(hs * aw_s[hd:hd + 1, :], axis=1, keepdims=True))
        dcols.append(jnp.sum(hs * aw_d[hd:hd + 1, :], axis=1, keepdims=True))
    return (jnp.concatenate(acols, axis=1),
            jnp.concatenate(dcols, axis=1))


def _proj_body(x_ref, w_ref, aws_ref, awd_ref, h_ref, as_ref, ad_ref):
    x = x_ref[0]
    w = w_ref[...]
    h = jnp.dot(x, w, preferred_element_type=jnp.float32)
    h_ref[0] = h
    as_ref[0], ad_ref[0] = _head_logits(h, aws_ref[...], awd_ref[...])


def _attn_body(lo_ref, hi_ref, posdt_ref, posnd_ref, h_ref, as_ref, adt_ref,
               b_ref, w_ref, p0_ref, p1_ref, *o_refs, fuse_fc):
    mk_ref = o_refs[-1]
    o_refs = o_refs[:-1]
    b = pl.program_id(0)
    j = pl.program_id(1)
    pos_dt = posdt_ref[0]
    sq_d = jnp.sum(pos_dt * pos_dt, axis=0, keepdims=True)
    ad_row = adt_ref[0]
    lo = lo_ref[b, j]
    hi = hi_ref[b, j]

    def chunk_mask(k):
        off = k * _SBLK
        pos_s = posnd_ref[0, pl.ds(off, _SBLK), :]
        sq_s = jnp.sum(pos_s * pos_s, axis=1, keepdims=True)
        dots = jnp.dot(pos_s, pos_dt, preferred_element_type=jnp.float32)
        d2 = sq_s + sq_d - 2.0 * dots
        return d2 < _RADIUS_SQ

    def masked_exp(k, mask):
        as_chunk = as_ref[0, pl.ds(k * _SBLK, _SBLK), :]
        exs = []
        for hd in range(_HEADS):
            e = as_chunk[:, hd:hd + 1] + ad_row[hd:hd + 1, :]
            e = jnp.where(e >= 0, e, 0.2 * e)
            exs.append(jnp.where(mask, jnp.exp(e), 0.0))
        return exs

    def den_chunk(k, carry):
        ls, lo1, hi1 = carry
        mask = chunk_mask(k)
        m01 = mask.astype(jnp.float32)
        mk_ref[pl.ds(k * _SBLK, _SBLK), :] = m01
        live = jnp.max(m01) > 0.0
        exs = masked_exp(k, mask)
        nls = tuple(ls[hd] + jnp.sum(exs[hd], axis=0, keepdims=True)
                    for hd in range(_HEADS))
        lo1 = jnp.where(live, jnp.minimum(lo1, k), lo1)
        hi1 = jnp.where(live, jnp.maximum(hi1, k), hi1)
        return nls, lo1, hi1

    ls, lo1, hi1 = jax.lax.fori_loop(
        lo, hi + 1, den_chunk,
        (tuple(jnp.zeros((1, _DBLK), jnp.float32) for _ in range(_HEADS)),
         jnp.int32(_NSB), jnp.int32(-1)))
    dens = [ls[hd] + 1e-16 for hd in range(_HEADS)]

    def agg_chunk(k, accs):
        off = k * _SBLK
        exs = masked_exp(k, mk_ref[pl.ds(off, _SBLK), :] > 0.0)
        h_chunk = h_ref[0, pl.ds(off, _SBLK), :]
        na = []
        for hd in range(_HEADS):
            alpha = exs[hd] / dens[hd]
            hc = h_chunk[:, hd * _HID:(hd + 1) * _HID]
            contrib = jax.lax.dot_general(
                hc, alpha, (((0,), (0,)), ((), ())),
                preferred_element_type=jnp.float32)
            na.append(accs[hd] + contrib)
        return tuple(na)

    accs = jax.lax.fori_loop(
        lo1, hi1 + 1, agg_chunk,
        tuple(jnp.zeros((_HID, _DBLK), jnp.float32) for _ in range(_HEADS)))

    out = jnp.concatenate(
        [jnp.transpose(accs[hd]) for hd in range(_HEADS)],
        axis=1)
    out = jnp.maximum(out + b_ref[...], 0.0)
    if fuse_fc:
        y = jnp.dot(out, w_ref[...],
                    preferred_element_type=jnp.float32) + p0_ref[...]
        o_refs[0][0] = y
    else:
        h2 = jnp.dot(out, w_ref[...], preferred_element_type=jnp.float32)
        o_refs[0][0] = h2
        o_refs[1][0], o_refs[2][0] = _head_logits(h2, p0_ref[...], p1_ref[...])


def _proj(x, w, aw_s, aw_d):
    cin = x.shape[-1]
    return pl.pallas_call(
        _proj_body,
        grid=(_B, _NP // _PBLK),
        in_specs=[
            pl.BlockSpec((1, _PBLK, cin), lambda b, j: (b, j, 0)),
            pl.BlockSpec((cin, _FEAT), lambda b, j: (0, 0)),
            pl.BlockSpec((_HEADS, _HID), lambda b, j: (0, 0)),
            pl.BlockSpec((_HEADS, _HID), lambda b, j: (0, 0)),
        ],
        out_specs=[
            pl.BlockSpec((1, _PBLK, _FEAT), lambda b, j: (b, j, 0)),
            pl.BlockSpec((1, _PBLK, _HEADS), lambda b, j: (b, j, 0)),
            pl.BlockSpec((1, _PBLK, _HEADS), lambda b, j: (b, j, 0)),
        ],
        out_shape=[
            jax.ShapeDtypeStruct((_B, _NP, _FEAT), jnp.float32),
            jax.ShapeDtypeStruct((_B, _NP, _HEADS), jnp.float32),
            jax.ShapeDtypeStruct((_B, _NP, _HEADS), jnp.float32),
        ],
        compiler_params=pltpu.CompilerParams(
            dimension_semantics=("parallel", "parallel")),
    )(x, w, aw_s, aw_d)


def _attn(lo, hi, pos_t, pos_nd, h, as_, ad_t, bias, w, p0, p1, fuse_fc):
    if fuse_fc:
        wshape = (_FEAT, _OUT_CH)
        p0shape = (1, _OUT_CH)
        p1shape = (1, _OUT_CH)
        out_specs = pl.BlockSpec((1, _DBLK, _OUT_CH),
                                 lambda b, j, lo, hi: (b, j, 0))
        out_shape = jax.ShapeDtypeStruct((_B, _NP, _OUT_CH), jnp.float32)
    else:
        wshape = (_FEAT, _FEAT)
        p0shape = (_HEADS, _HID)
        p1shape = (_HEADS, _HID)
        out_specs = [
            pl.BlockSpec((1, _DBLK, _FEAT), lambda b, j, lo, hi: (b, j, 0)),
            pl.BlockSpec((1, _DBLK, _HEADS), lambda b, j, lo, hi: (b, j, 0)),
            pl.BlockSpec((1, _DBLK, _HEADS), lambda b, j, lo, hi: (b, j, 0)),
        ]
        out_shape = [
            jax.ShapeDtypeStruct((_B, _NP, _FEAT), jnp.float32),
            jax.ShapeDtypeStruct((_B, _NP, _HEADS), jnp.float32),
            jax.ShapeDtypeStruct((_B, _NP, _HEADS), jnp.float32),
        ]
    return pl.pallas_call(
        functools.partial(_attn_body, fuse_fc=fuse_fc),
        grid_spec=pltpu.PrefetchScalarGridSpec(
            num_scalar_prefetch=2,
            grid=(_B, _NDB),
            in_specs=[
                pl.BlockSpec((1, 3, _DBLK), lambda b, j, lo, hi: (b, 0, j)),
                pl.BlockSpec((1, _NP, 3), lambda b, j, lo, hi: (b, 0, 0)),
                pl.BlockSpec((1, _NP, _FEAT), lambda b, j, lo, hi: (b, 0, 0)),
                pl.BlockSpec((1, _NP, _HEADS), lambda b, j, lo, hi: (b, 0, 0)),
                pl.BlockSpec((1, _HEADS, _DBLK), lambda b, j, lo, hi: (b, 0, j)),
                pl.BlockSpec((1, _FEAT), lambda b, j, lo, hi: (0, 0)),
                pl.BlockSpec(wshape, lambda b, j, lo, hi: (0, 0)),
                pl.BlockSpec(p0shape, lambda b, j, lo, hi: (0, 0)),
                pl.BlockSpec(p1shape, lambda b, j, lo, hi: (0, 0)),
            ],
            out_specs=out_specs,
            scratch_shapes=[pltpu.VMEM((_NP, _DBLK), jnp.float32)],
        ),
        out_shape=out_shape,
        compiler_params=pltpu.CompilerParams(
            dimension_semantics=("parallel", "arbitrary")),
    )(lo, hi, pos_t, pos_nd, h, as_, ad_t, bias, w, p0, p1)


def _chunk_ranges(pos_t):
    xp = pos_t[:, 0, :]
    cs_min = xp[:, 0::_SBLK]
    cs_max = xp[:, _SBLK - 1::_SBLK]
    xd_min = xp[:, 0::_DBLK]
    xd_max = xp[:, _DBLK - 1::_DBLK]
    lo = jax.vmap(lambda a, v: jnp.searchsorted(a, v, side='left'))(
        cs_max, xd_min - _TH).astype(jnp.int32)
    hi = (jax.vmap(lambda a, v: jnp.searchsorted(a, v, side='right'))(
        cs_min, xd_max + _TH) - 1).astype(jnp.int32)
    return lo, hi


def kernel(pos, pos_non_manifold, W1, a1_src, a1_dst, b1,
           W2, a2_src, a2_dst, b2, W_fc, b_fc):
    pos_t = jnp.concatenate([pos, pos_non_manifold], axis=2)
    perm = jnp.argsort(pos_t[:, 0, :], axis=1)
    inv = jnp.argsort(perm, axis=1)
    pos_t = jnp.take_along_axis(pos_t, perm[:, None, :], axis=2)
    pos_t = jnp.pad(pos_t, ((0, 0), (0, 0), (0, _NP - _N)),
                    constant_values=_PAD_VAL)
    pos_nd = pos_t.transpose(0, 2, 1)
    pos_nd0 = pos_nd.at[:, _N:, :].set(0.0)
    lo, hi = _chunk_ranges(pos_t)

    b1r = b1.reshape(1, _FEAT)
    b2r = b2.reshape(1, _FEAT)
    bfc = b_fc.reshape(1, _OUT_CH)

    h, as_, ad = _proj(pos_nd0, W1, a1_src, a1_dst)
    h2, as2, ad2 = _attn(lo, hi, pos_t, pos_nd, h, as_,
                         ad.transpose(0, 2, 1), b1r, W2,
                         a2_src, a2_dst, fuse_fc=False)
    y = _attn(lo, hi, pos_t, pos_nd, h2, as2, ad2.transpose(0, 2, 1),
              b2r, W_fc, bfc, bfc, fuse_fc=True)

    y = jnp.take_along_axis(y[:, :_N], inv[:, :, None], axis=1)
    return y[:, _N_NM:_N].reshape(_B, _OUT_CH, _N_SURF)


# speedup vs baseline: 1.1924x; 1.0174x over previous
import functools

import jax
import jax.numpy as jnp
from jax.experimental import pallas as pl
from jax.experimental.pallas import tpu as pltpu

_B = 2
_N_SURF = 3000
_N_NM = 1000
_N = _N_SURF + _N_NM
_NP = 4096
_HEADS = 4
_HID = 64
_FEAT = _HEADS * _HID
_OUT_CH = 2
_PBLK = 1024
_DBLK = 512
_SBLK = 256
_NDB = _NP // _DBLK
_NSB = _NP // _SBLK
_RADIUS = 0.05
_RADIUS_SQ = float(0.0025)
_TH = 0.17
_PAD_VAL = 100.0


def _head_logits(h, aw_s, aw_d):
    acols, dcols = [], []
    for hd in range(_HEADS):
        hs = h[:, hd * _HID:(hd + 1) * _HID]
        acols.append(jnp.sum(hs * aw_s[hd:hd + 1, :], axis=1, keepdims=True))
        dcols.append(jnp.sum(hs * aw_d[hd:hd + 1, :], axis=1, keepdims=True))
    return (jnp.concatenate(acols, axis=1),
            jnp.concatenate(dcols, axis=1))


def _proj_body(x_ref, w_ref, aws_ref, awd_ref, h_ref, as_ref, ad_ref):
    x = x_ref[0]
    w = w_ref[...]
    h = jnp.dot(x, w, preferred_element_type=jnp.float32)
    h_ref[0] = h
    as_ref[0], ad_ref[0] = _head_logits(h, aws_ref[...], awd_ref[...])


def _attn_body(lo_ref, hi_ref, posdt_ref, posnd_ref, h_ref, as_ref, adt_ref,
               b_ref, w_ref, p0_ref, p1_ref, *o_refs, fuse_fc):
    mk_ref = o_refs[-1]
    o_refs = o_refs[:-1]
    b = pl.program_id(0)
    j = pl.program_id(1)
    pos_dt = posdt_ref[0]
    sq_d = jnp.sum(pos_dt * pos_dt, axis=0, keepdims=True)
    ad_row = adt_ref[0]
    lo = lo_ref[b, j]
    hi = hi_ref[b, j]

    def chunk_mask(k):
        off = k * _SBLK
        pos_s = posnd_ref[0, pl.ds(off, _SBLK), :]
        sq_s = jnp.sum(pos_s * pos_s, axis=1, keepdims=True)
        dots = jnp.dot(pos_s, pos_dt, preferred_element_type=jnp.float32)
        d2 = sq_s + sq_d - 2.0 * dots
        return d2 < _RADIUS_SQ

    def masked_exp(k, mask):
        as_chunk = as_ref[0, pl.ds(k * _SBLK, _SBLK), :]
        exs = []
        for hd in range(_HEADS):
            e = as_chunk[:, hd:hd + 1] + ad_row[hd:hd + 1, :]
            e = jnp.where(e >= 0, e, 0.2 * e)
            exs.append(jnp.where(mask, jnp.exp(e), 0.0))
        return exs

    def den_chunk(k, carry):
        ls, lo1, hi1 = carry
        mask = chunk_mask(k)
        m01 = mask.astype(jnp.float32)
        mk_ref[pl.ds(k * _SBLK, _SBLK), :] = m01
        live = jnp.max(m01) > 0.0
        exs = masked_exp(k, mask)
        nls = tuple(ls[hd] + jnp.sum(exs[hd], axis=0, keepdims=True)
                    for hd in range(_HEADS))
        lo1 = jnp.where(live, jnp.minimum(lo1, k), lo1)
        hi1 = jnp.where(live, jnp.maximum(hi1, k), hi1)
        return nls, lo1, hi1

    ls, lo1, hi1 = jax.lax.fori_loop(
        lo, hi + 1, den_chunk,
        (tuple(jnp.zeros((1, _DBLK), jnp.float32) for _ in range(_HEADS)),
         jnp.int32(_NSB), jnp.int32(-1)))
    dens = [ls[hd] + 1e-16 for hd in range(_HEADS)]

    def agg_chunk(k, accs):
        off = k * _SBLK
        exs = masked_exp(k, mk_ref[pl.ds(off, _SBLK), :] > 0.0)
        h_chunk = h_ref[0, pl.ds(off, _SBLK), :]
        na = []
        for hd in range(_HEADS):
            alpha = exs[hd] / dens[hd]
            hc = h_chunk[:, hd * _HID:(hd + 1) * _HID]
            contrib = jax.lax.dot_general(
                hc, alpha, (((0,), (0,)), ((), ())),
                preferred_element_type=jnp.float32)
            na.append(accs[hd] + contrib)
        return tuple(na)

    accs = jax.lax.fori_loop(
        lo1, hi1 + 1, agg_chunk,
        tuple(jnp.zeros((_HID, _DBLK), jnp.float32) for _ in range(_HEADS)))

    out = jnp.concatenate(
        [jnp.transpose(accs[hd]) for hd in range(_HEADS)],
        axis=1)
    out = jnp.maximum(out + b_ref[...], 0.0)
    if fuse_fc:
        y = jnp.dot(out, w_ref[...],
                    preferred_element_type=jnp.float32) + p0_ref[...]
        o_refs[0][0] = y
    else:
        h2 = jnp.dot(out, w_ref[...], preferred_element_type=jnp.float32)
        o_refs[0][0] = h2
        o_refs[1][0], o_refs[2][0] = _head_logits(h2, p0_ref[...], p1_ref[...])


def _proj(x, w, aw_s, aw_d):
    cin = x.shape[-1]
    return pl.pallas_call(
        _proj_body,
        grid=(_B, _NP // _PBLK),
        in_specs=[
            pl.BlockSpec((1, _PBLK, cin), lambda b, j: (b, j, 0)),
            pl.BlockSpec((cin, _FEAT), lambda b, j: (0, 0)),
            pl.BlockSpec((_HEADS, _HID), lambda b, j: (0, 0)),
            pl.BlockSpec((_HEADS, _HID), lambda b, j: (0, 0)),
        ],
        out_specs=[
            pl.BlockSpec((1, _PBLK, _FEAT), lambda b, j: (b, j, 0)),
            pl.BlockSpec((1, _PBLK, _HEADS), lambda b, j: (b, j, 0)),
            pl.BlockSpec((1, _PBLK, _HEADS), lambda b, j: (b, j, 0)),
        ],
        out_shape=[
            jax.ShapeDtypeStruct((_B, _NP, _FEAT), jnp.float32),
            jax.ShapeDtypeStruct((_B, _NP, _HEADS), jnp.float32),
            jax.ShapeDtypeStruct((_B, _NP, _HEADS), jnp.float32),
        ],
        compiler_params=pltpu.CompilerParams(
            dimension_semantics=("parallel", "parallel")),
    )(x, w, aw_s, aw_d)


def _attn(lo, hi, pos_t, pos_nd, h, as_, ad_t, bias, w, p0, p1, fuse_fc):
    if fuse_fc:
        wshape = (_FEAT, _OUT_CH)
        p0shape = (1, _OUT_CH)
        p1shape = (1, _OUT_CH)
        out_specs = pl.BlockSpec((1, _DBLK, _OUT_CH),
                                 lambda b, j, lo, hi: (b, j, 0))
        out_shape = jax.ShapeDtypeStruct((_B, _NP, _OUT_CH), jnp.float32)
    else:
        wshape = (_FEAT, _FEAT)
        p0shape = (_HEADS, _HID)
        p1shape = (_HEADS, _HID)
        out_specs = [
            pl.BlockSpec((1, _DBLK, _FEAT), lambda b, j, lo, hi: (b, j, 0)),
            pl.BlockSpec((1, _DBLK, _HEADS), lambda b, j, lo, hi: (b, j, 0)),
            pl.BlockSpec((1, _DBLK, _HEADS), lambda b, j, lo, hi: (b, j, 0)),
        ]
        out_shape = [
            jax.ShapeDtypeStruct((_B, _NP, _FEAT), jnp.float32),
            jax.ShapeDtypeStruct((_B, _NP, _HEADS), jnp.float32),
            jax.ShapeDtypeStruct((_B, _NP, _HEADS), jnp.float32),
        ]
    return pl.pallas_call(
        functools.partial(_attn_body, fuse_fc=fuse_fc),
        grid_spec=pltpu.PrefetchScalarGridSpec(
            num_scalar_prefetch=2,
            grid=(_B, _NDB),
            in_specs=[
                pl.BlockSpec((1, 3, _DBLK), lambda b, j, lo, hi: (b, 0, j)),
                pl.BlockSpec((1, _NP, 3), lambda b, j, lo, hi: (b, 0, 0)),
                pl.BlockSpec((1, _NP, _FEAT), lambda b, j, lo, hi: (b, 0, 0)),
                pl.BlockSpec((1, _NP, _HEADS), lambda b, j, lo, hi: (b, 0, 0)),
                pl.BlockSpec((1, _HEADS, _DBLK), lambda b, j, lo, hi: (b, 0, j)),
                pl.BlockSpec((1, _FEAT), lambda b, j, lo, hi: (0, 0)),
                pl.BlockSpec(wshape, lambda b, j, lo, hi: (0, 0)),
                pl.BlockSpec(p0shape, lambda b, j, lo, hi: (0, 0)),
                pl.BlockSpec(p1shape, lambda b, j, lo, hi: (0, 0)),
            ],
            out_specs=out_specs,
            scratch_shapes=[pltpu.VMEM((_NP, _DBLK), jnp.float32)],
        ),
        out_shape=out_shape,
        compiler_params=pltpu.CompilerParams(
            dimension_semantics=("parallel", "arbitrary")),
    )(lo, hi, pos_t, pos_nd, h, as_, ad_t, bias, w, p0, p1)


def _chunk_ranges(pos_t):
    xp = pos_t[:, 0, :]
    cs_min = xp[:, 0::_SBLK]
    cs_max = xp[:, _SBLK - 1::_SBLK]
    xd_min = xp[:, 0::_DBLK]
    xd_max = xp[:, _DBLK - 1::_DBLK]
    lo = jax.vmap(lambda a, v: jnp.searchsorted(a, v, side='left'))(
        cs_max, xd_min - _TH).astype(jnp.int32)
    hi = (jax.vmap(lambda a, v: jnp.searchsorted(a, v, side='right'))(
        cs_min, xd_max + _TH) - 1).astype(jnp.int32)
    return lo, hi


def kernel(pos, pos_non_manifold, W1, a1_src, a1_dst, b1,
           W2, a2_src, a2_dst, b2, W_fc, b_fc):
    pos_t = jnp.concatenate([pos, pos_non_manifold], axis=2)
    perm = jnp.argsort(pos_t[:, 0, :], axis=1)
    inv = jnp.argsort(perm, axis=1)
    pos_t = jnp.take_along_axis(pos_t, perm[:, None, :], axis=2)
    pos_t = jnp.pad(pos_t, ((0, 0), (0, 0), (0, _NP - _N)),
                    constant_values=_PAD_VAL)
    pos_nd = pos_t.transpose(0, 2, 1)
    pos_nd0 = pos_nd.at[:, _N:, :].set(0.0)
    lo, hi = _chunk_ranges(pos_t)

    b1r = b1.reshape(1, _FEAT)
    b2r = b2.reshape(1, _FEAT)
    bfc = b_fc.reshape(1, _OUT_CH)

    h, as_, ad = _proj(pos_nd0, W1, a1_src, a1_dst)
    h2, as2, ad2 = _attn(lo, hi, pos_t, pos_nd, h, as_,
                         ad.transpose(0, 2, 1), b1r, W2,
                         a2_src, a2_dst, fuse_fc=False)
    y = _attn(lo, hi, pos_t, pos_nd, h2, as2, ad2.transpose(0, 2, 1),
              b2r, W_fc, bfc, bfc, fuse_fc=True)

    y = jnp.take_along_axis(y[:, :_N], inv[:, :, None], axis=1)
    return y[:, _N_NM:_N].reshape(_B, _OUT_CH, _N_SURF)
